# Initial kernel scaffold; baseline (speedup 1.0000x reference)
import jax
import jax.numpy as jnp
from jax.experimental import pallas as pl


def kernel(X, Y):
    raise NotImplementedError("write your pallas kernel here")


# speedup vs baseline: 35.1406x; 35.1406x over previous
import functools

import jax
import jax.numpy as jnp
from jax.experimental import pallas as pl
from jax.experimental.pallas import tpu as pltpu

B, N, D_FEAT = 4, 512, 128
N0 = N - 1
N1 = N // 2
BIG = 1e30


def _betti_kernel(x_ref, out_ref, dm_ref):
    ri = jax.lax.broadcasted_iota(jnp.int32, (N, N), 0)
    ci = jax.lax.broadcasted_iota(jnp.int32, (N, N), 1)

    nn_vals = []
    for b in range(B):
        pts = x_ref[b]
        nrm = jnp.sqrt(jnp.sum(pts * pts, axis=0, keepdims=True))
        pts = pts / jnp.maximum(nrm, 1e-12)
        g = jax.lax.dot_general(
            pts, pts, (((1,), (1,)), ((), ())),
            preferred_element_type=jnp.float32,
            precision=jax.lax.Precision.HIGHEST,
        )
        sq = jnp.sum(pts * pts, axis=1, keepdims=True)
        sqc = jnp.min(jnp.where(ri == ci, g, BIG), axis=0, keepdims=True)
        d2 = jnp.maximum(sq + sqc - 2.0 * g, 0.0)
        dm = jnp.sqrt(d2 + 1e-12)
        dm = jnp.where(ri == ci, BIG, dm)
        dm_ref[b, :, :] = dm

        m1 = jnp.min(dm, axis=1, keepdims=True)
        jmin = jnp.min(jnp.where(dm == m1, ci, N), axis=1, keepdims=True)
        m2 = jnp.min(jnp.where(ci == jmin, BIG, dm), axis=1, keepdims=True)
        diff = m2 - m1
        rrow = jax.lax.broadcasted_iota(jnp.int32, (N, 1), 0)
        nn_vals.append(jnp.sum(jnp.where(rrow < N1, diff * diff, 0.0)))

    ci_b = jax.lax.broadcasted_iota(jnp.int32, (B, N), 1)
    md0 = dm_ref[:, 0, :]

    def step(_, carry):
        md, acc = carry
        m = jnp.min(md, axis=1, keepdims=True)
        tagged = jnp.where(md == m, ci_b, N)
        jvec = jnp.min(tagged, axis=1, keepdims=True)
        rows = []
        for b in range(B):
            j = jnp.min(tagged[b : b + 1, :])
            rows.append(dm_ref[b, pl.ds(j, 1), :])
        row = jnp.concatenate(rows, axis=0)
        md = jnp.where(ci_b == jvec, BIG, jnp.minimum(md, row))
        return md, acc + m * m

    _, acc = jax.lax.fori_loop(
        0, N - 1, step, (md0, jnp.zeros((B, 1), jnp.float32))
    )

    for b in range(B):
        mst_sq = jnp.sum(acc[b : b + 1, :])
        out_ref[b] = jnp.float32(N0 + N1) - mst_sq - nn_vals[b]


@jax.jit
def kernel(X, Y):
    del Y
    return pl.pallas_call(
        _betti_kernel,
        out_shape=jax.ShapeDtypeStruct((B,), jnp.float32),
        out_specs=pl.BlockSpec(memory_space=pltpu.SMEM),
        scratch_shapes=[pltpu.VMEM((B, N, N), jnp.float32)],
    )(X)


# speedup vs baseline: 48.5617x; 1.3819x over previous
import functools

import jax
import jax.numpy as jnp
from jax.experimental import pallas as pl
from jax.experimental.pallas import tpu as pltpu

B, N, D_FEAT = 4, 512, 128
N0 = N - 1
N1 = N // 2
BIG = 1e30
IDX_MASK = N - 1
BIGKEY = 0x7F7FFFFF


def _betti_kernel(x_ref, out_ref, dm_ref):
    ri = jax.lax.broadcasted_iota(jnp.int32, (N, N), 0)
    ci = jax.lax.broadcasted_iota(jnp.int32, (N, N), 1)

    nn_vals = []
    for b in range(B):
        pts = x_ref[b]
        nrm = jnp.sqrt(jnp.sum(pts * pts, axis=0, keepdims=True))
        pts = pts / jnp.maximum(nrm, 1e-12)
        g = jax.lax.dot_general(
            pts, pts, (((1,), (1,)), ((), ())),
            preferred_element_type=jnp.float32,
            precision=jax.lax.Precision.HIGHEST,
        )
        sq = jnp.sum(pts * pts, axis=1, keepdims=True)
        sqc = jnp.min(jnp.where(ri == ci, g, BIG), axis=0, keepdims=True)
        d2 = jnp.maximum(sq + sqc - 2.0 * g, 0.0)
        dm = jnp.sqrt(d2 + 1e-12)
        dm = jnp.where(ri == ci, BIG, dm)
        kd = (jax.lax.bitcast_convert_type(dm, jnp.int32) & ~IDX_MASK) | ci
        dm_ref[b, :, :] = kd

        m1 = jnp.min(dm, axis=1, keepdims=True)
        jmin = jnp.min(jnp.where(dm == m1, ci, N), axis=1, keepdims=True)
        m2 = jnp.min(jnp.where(ci == jmin, BIG, dm), axis=1, keepdims=True)
        diff = m2 - m1
        rrow = jax.lax.broadcasted_iota(jnp.int32, (N, 1), 0)
        nn_vals.append(jnp.sum(jnp.where(rrow < N1, diff * diff, 0.0)))

    ci_b = jax.lax.broadcasted_iota(jnp.int32, (B, N), 1)
    md0 = dm_ref[:, 0, :]

    def step(_, carry):
        md, acc = carry
        kmin = jnp.min(md, axis=1, keepdims=True)
        jvec = kmin & IDX_MASK
        w = jax.lax.bitcast_convert_type(kmin & ~IDX_MASK, jnp.float32)
        rows = []
        for b in range(B):
            j = jnp.min(kmin[b : b + 1, :]) & IDX_MASK
            rows.append(dm_ref[b, pl.ds(j, 1), :])
        row = jnp.concatenate(rows, axis=0)
        md = jnp.where(ci_b == jvec, BIGKEY, jnp.minimum(md, row))
        return md, acc + w * w

    _, acc = jax.lax.fori_loop(
        0, N - 1, step, (md0, jnp.zeros((B, 1), jnp.float32))
    )

    for b in range(B):
        mst_sq = jnp.sum(acc[b : b + 1, :])
        out_ref[b] = jnp.float32(N0 + N1) - mst_sq - nn_vals[b]


@jax.jit
def kernel(X, Y):
    del Y
    return pl.pallas_call(
        _betti_kernel,
        out_shape=jax.ShapeDtypeStruct((B,), jnp.float32),
        out_specs=pl.BlockSpec(memory_space=pltpu.SMEM),
        scratch_shapes=[pltpu.VMEM((B, N, N), jnp.int32)],
    )(X)
